# baseline (speedup 1.0000x reference)
import functools

import jax
import jax.numpy as jnp
from jax import lax
from jax.experimental import pallas as pl
from jax.experimental.pallas import tpu as pltpu
from jax.experimental.pallas import tpu_sc as plsc

D_EMB = 768
SEQ_LEN = 77
BATCH = 1024
NW = 32
RC = 64
CPS = BATCH // RC
NCHUNK = CPS * SEQ_LEN
LANES = 16


def _make_sc_embed():
    mesh = plsc.VectorSubcoreMesh(core_axis_name="c", subcore_axis_name="s")

    @functools.partial(
        pl.kernel,
        mesh=mesh,
        out_type=jax.ShapeDtypeStruct((BATCH * SEQ_LEN, D_EMB), jnp.float32),
        scratch_types=[
            pltpu.VMEM((RC,), jnp.int32),
            pltpu.VMEM((RC,), jnp.int32),
            pltpu.VMEM((RC, D_EMB), jnp.float32),
            pltpu.VMEM((1, D_EMB), jnp.float32),
            pltpu.SemaphoreType.DMA,
        ],
    )
    def k(tok_hbm, orow_hbm, table_hbm, pos_hbm, out_hbm,
          idx_v, orow_v, rows_v, pos_v, sem):
        wid = lax.axis_index("s") * 2 + lax.axis_index("c")

        def body(i, carry):
            c = wid + i * NW

            @pl.when(c < NCHUNK)
            def _():
                base = c * RC
                s_pos = c // CPS
                pltpu.sync_copy(tok_hbm.at[pl.ds(base, RC)], idx_v)
                pltpu.sync_copy(orow_hbm.at[pl.ds(base, RC)], orow_v)
                pltpu.sync_copy(pos_hbm.at[pl.ds(s_pos, 1)], pos_v)
                pltpu.async_copy(table_hbm.at[idx_v], rows_v, sem).wait()

                def add_col(kk, carry2):
                    pv = pos_v[0, pl.ds(kk * LANES, LANES)]

                    def add_row(r, carry3):
                        rows_v[r, pl.ds(kk * LANES, LANES)] = (
                            rows_v[r, pl.ds(kk * LANES, LANES)] + pv)
                        return carry3

                    return lax.fori_loop(0, RC, add_row, carry2)

                lax.fori_loop(0, D_EMB // LANES, add_col, 0)
                pltpu.async_copy(rows_v, out_hbm.at[orow_v], sem).wait()

            return carry

        lax.fori_loop(0, (NCHUNK + NW - 1) // NW, body, 0)

    return k


_sc_embed = _make_sc_embed()


def kernel(tokens, embedding_table, positional_embedding):
    tokens_sm = tokens.T.reshape(-1).astype(jnp.int32)
    b_ids = jnp.tile(jnp.arange(BATCH, dtype=jnp.int32), SEQ_LEN)
    s_ids = jnp.repeat(jnp.arange(SEQ_LEN, dtype=jnp.int32), BATCH)
    out_rows = b_ids * SEQ_LEN + s_ids
    out_flat = _sc_embed(tokens_sm, out_rows, embedding_table,
                         positional_embedding)
    return out_flat.reshape(BATCH, SEQ_LEN, D_EMB)


# speedup vs baseline: 2.3118x; 2.3118x over previous
import functools

import jax
import jax.numpy as jnp
from jax import lax
from jax.experimental import pallas as pl
from jax.experimental.pallas import tpu as pltpu
from jax.experimental.pallas import tpu_sc as plsc

D_EMB = 768
SEQ_LEN = 77
BATCH = 1024
NW = 32
RC = 32
CPS = BATCH // RC
NCHUNK = CPS * SEQ_LEN
NBUF = 3
NITER = 78
NCHUNK_PAD = NITER * NW
LANES = 16
BLK = 12
NKB = D_EMB // (BLK * LANES)


def _make_sc_embed():
    mesh = plsc.VectorSubcoreMesh(core_axis_name="c", subcore_axis_name="s")

    @functools.partial(
        pl.kernel,
        mesh=mesh,
        out_type=jax.ShapeDtypeStruct((BATCH * SEQ_LEN, D_EMB), jnp.float32),
        scratch_types=(
            [pltpu.VMEM((2, RC), jnp.int32) for _ in range(NBUF)]
            + [pltpu.VMEM((1, D_EMB), jnp.float32) for _ in range(NBUF)]
            + [pltpu.VMEM((RC, D_EMB), jnp.float32) for _ in range(NBUF)]
            + [pltpu.SemaphoreType.DMA for _ in range(3 * NBUF)]
        ),
    )
    def k(io_hbm, table_hbm, pos_hbm, out_hbm,
          io0, io1, io2, pos0, pos1, pos2, rows0, rows1, rows2,
          g0, g1, g2, s0, s1, s2, p0, p1, p2):
        io_b = [io0, io1, io2]
        pos_b = [pos0, pos1, pos2]
        rows_b = [rows0, rows1, rows2]
        gsem = [g0, g1, g2]
        ssem = [s0, s1, s2]
        psem = [p0, p1, p2]

        wid = lax.axis_index("s") * 2 + lax.axis_index("c")

        def stage(nb, c):
            pltpu.sync_copy(io_hbm.at[c], io_b[nb])
            pltpu.async_copy(pos_hbm.at[pl.ds(c // CPS, 1)], pos_b[nb],
                             psem[nb])
            pltpu.async_copy(table_hbm.at[io_b[nb].at[0]], rows_b[nb],
                             gsem[nb])

        def wait_scatter(nb):
            pltpu.make_async_copy(rows_b[nb], out_hbm.at[io_b[nb].at[1]],
                                  ssem[nb]).wait()

        def add_pos(b):
            rows = rows_b[b]
            pos = pos_b[b]
            for kb in range(NKB):
                pvs = [pos[0, pl.ds(kb * BLK * LANES + j * LANES, LANES)]
                       for j in range(BLK)]

                def row_body(r, carry, _kb=kb, _pvs=pvs):
                    for j in range(BLK):
                        col = _kb * BLK * LANES + j * LANES
                        rows[r, pl.ds(col, LANES)] = (
                            rows[r, pl.ds(col, LANES)] + _pvs[j])
                    return carry

                lax.fori_loop(0, RC, row_body, 0, unroll=2)

        stage(0, wid)

        def body(j3, carry):
            for b in range(NBUF):
                j = j3 * NBUF + b
                c = wid + j * NW
                nb = (b + 1) % NBUF

                def prefetch():
                    if b == 2:
                        wait_scatter(nb)
                    else:
                        @pl.when(j3 >= 1)
                        def _():
                            wait_scatter(nb)
                    stage(nb, c + NW)

                if b == NBUF - 1:
                    @pl.when(j3 < NITER // NBUF - 1)
                    def _():
                        prefetch()
                else:
                    prefetch()

                pltpu.make_async_copy(table_hbm.at[io_b[b].at[0]], rows_b[b],
                                      gsem[b]).wait()
                pltpu.make_async_copy(pos_hbm.at[pl.ds(c // CPS, 1)],
                                      pos_b[b], psem[b]).wait()
                add_pos(b)
                pltpu.async_copy(rows_b[b], out_hbm.at[io_b[b].at[1]],
                                 ssem[b])
            return carry

        lax.fori_loop(0, NITER // NBUF, body, 0)

        for nb in range(NBUF):
            wait_scatter(nb)

    return k


_sc_embed = _make_sc_embed()


def kernel(tokens, embedding_table, positional_embedding):
    tok_sm = tokens.T.reshape(-1).astype(jnp.int32)
    b_ids = jnp.tile(jnp.arange(BATCH, dtype=jnp.int32), SEQ_LEN)
    s_ids = jnp.repeat(jnp.arange(SEQ_LEN, dtype=jnp.int32), BATCH)
    out_rows = b_ids * SEQ_LEN + s_ids

    npad = (NCHUNK_PAD - NCHUNK) * RC
    tok_pad = jnp.concatenate([tok_sm, tok_sm[:npad]])
    orow_pad = jnp.concatenate([out_rows, out_rows[:npad]])
    io = jnp.stack([tok_pad.reshape(NCHUNK_PAD, RC),
                    orow_pad.reshape(NCHUNK_PAD, RC)], axis=1)
    pos_pad = jnp.concatenate(
        [positional_embedding, positional_embedding[:1]])

    out_flat = _sc_embed(io, embedding_table, pos_pad)
    return out_flat.reshape(BATCH, SEQ_LEN, D_EMB)
